# baseline (speedup 1.0000x reference)
import jax
import jax.numpy as jnp
from jax import lax
from jax.experimental import pallas as pl
from jax.experimental.pallas import tpu as pltpu
from jax.experimental.pallas import tpu_sc as plsc

B = 16384
D = 1024
DH = D // 2
L = 16
NC = 2
NS = 16
NW = NC * NS
RPW = B // NW
R = 16
NBLK = RPW // R
NSLOT = 3


def _sc_body(c0, cond1, cond4, cond5, emb1, out,
             ebuf, edbuf, c1buf,
             c0b0, c0b1, c0b2, m4b0, m4b1, m4b2, m5b0, m5b1, m5b2,
             isem0, isem1, isem2, osem0, osem1, osem2):
    c0bs = (c0b0, c0b1, c0b2)
    m4bs = (m4b0, m4b1, m4b2)
    m5bs = (m5b0, m5b1, m5b2)
    isems = (isem0, isem1, isem2)
    osems = (osem0, osem1, osem2)

    cid = lax.axis_index("c")
    sid = lax.axis_index("s")
    wid = sid * NC + cid
    base = wid * RPW

    pltpu.sync_copy(emb1, ebuf)
    pltpu.sync_copy(cond1.at[pl.ds(base, RPW)], c1buf)
    for k in range(D // L):
        sl = pl.ds(k * L, L)
        edbuf[sl] = ebuf[1, sl] - ebuf[0, sl]

    def start_in(s, g):
        rb = base + g * R
        pltpu.async_copy(c0.at[pl.ds(rb, R)], c0bs[s], isems[s])
        pltpu.async_copy(cond4.at[pl.ds(rb, R)], m4bs[s], isems[s])
        pltpu.async_copy(cond5.at[pl.ds(rb, R)], m5bs[s], isems[s])

    def wait_in(s, g):
        rb = base + g * R
        pltpu.make_async_copy(c0.at[pl.ds(rb, R)], c0bs[s], isems[s]).wait()
        pltpu.make_async_copy(cond4.at[pl.ds(rb, R)], m4bs[s], isems[s]).wait()
        pltpu.make_async_copy(cond5.at[pl.ds(rb, R)], m5bs[s], isems[s]).wait()

    def start_out(s, g):
        rb = base + g * R
        pltpu.async_copy(c0bs[s], out.at[pl.ds(rb, R)], osems[s])

    def wait_out(s, g):
        rb = base + g * R
        pltpu.make_async_copy(c0bs[s], out.at[pl.ds(rb, R)], osems[s]).wait()

    def compute(s, g):
        c0r, m4r, m5r = c0bs[s], m4bs[s], m5bs[s]
        fvec = c1buf[pl.ds(g * R, R)].astype(jnp.float32)
        fs = [jnp.full((L,), fvec[i], jnp.float32) for i in range(R)]

        @plsc.parallel_loop(0, DH // L, unroll=2)
        def _first_half(j):
            sl = pl.ds(j * L, L)
            e0c = ebuf[0, sl]
            edc = edbuf[sl]
            for i in range(R):
                plsc.addupdate(c0r.at[i, sl], e0c + fs[i] * edc + m4r[i, sl])

        @plsc.parallel_loop(0, DH // L, unroll=2)
        def _second_half(j):
            sl2 = pl.ds(DH + j * L, L)
            sl = pl.ds(j * L, L)
            e0c = ebuf[0, sl2]
            edc = edbuf[sl2]
            for i in range(R):
                plsc.addupdate(c0r.at[i, sl2], e0c + fs[i] * edc + m5r[i, sl])

    def turn(g, b, head=False, tail=False):
        sn = (b + 1) % NSLOT
        if not head:
            wait_out(sn, g - 2)
        if not tail:
            start_in(sn, g + 1)
        wait_in(b, g)
        compute(b, g)
        start_out(b, g)

    start_in(0, 0)
    turn(0, 0, head=True)
    turn(1, 1, head=True)

    @pl.loop(0, (NBLK - 5) // NSLOT)
    def _main(t):
        g0 = 2 + t * NSLOT
        for d in range(NSLOT):
            turn(g0 + d, (2 + d) % NSLOT)

    turn(NBLK - 3, (NBLK - 3) % NSLOT)
    turn(NBLK - 2, (NBLK - 2) % NSLOT)
    turn(NBLK - 1, (NBLK - 1) % NSLOT, tail=True)
    wait_out((NBLK - 2) % NSLOT, NBLK - 2)
    wait_out((NBLK - 1) % NSLOT, NBLK - 1)


def kernel(c0, cond1, cond4, cond5, emb1):
    mesh = plsc.VectorSubcoreMesh(
        core_axis_name="c", subcore_axis_name="s",
        num_cores=NC, num_subcores=NS)
    f = pl.kernel(
        _sc_body,
        out_type=jax.ShapeDtypeStruct((B, D), jnp.float32),
        mesh=mesh,
        scratch_types=[
            pltpu.VMEM((2, D), jnp.float32),
            pltpu.VMEM((D,), jnp.float32),
            pltpu.VMEM((RPW,), jnp.int32),
            pltpu.VMEM((R, D), jnp.float32),
            pltpu.VMEM((R, D), jnp.float32),
            pltpu.VMEM((R, D), jnp.float32),
            pltpu.VMEM((R, DH), jnp.float32),
            pltpu.VMEM((R, DH), jnp.float32),
            pltpu.VMEM((R, DH), jnp.float32),
            pltpu.VMEM((R, DH), jnp.float32),
            pltpu.VMEM((R, DH), jnp.float32),
            pltpu.VMEM((R, DH), jnp.float32),
            pltpu.SemaphoreType.DMA,
            pltpu.SemaphoreType.DMA,
            pltpu.SemaphoreType.DMA,
            pltpu.SemaphoreType.DMA,
            pltpu.SemaphoreType.DMA,
            pltpu.SemaphoreType.DMA,
        ],
    )
    return f(c0, cond1, cond4, cond5, emb1)
